# baseline (speedup 1.0000x reference)
import functools

import jax
import jax.numpy as jnp
from jax.experimental import pallas as pl
from jax.experimental.pallas import tpu as pltpu

KS = 5
PAD = KS // 2


def _pair_kernel(x_ref, dww_ref, dwb_ref, wblk_ref, pwb_ref, out_ref, acc_ref,
                 *, H, W, HWp, R, n_chunks):
    f32 = jnp.float32
    bf16 = jnp.bfloat16
    CR = R // n_chunks
    Ro = out_ref.shape[0]

    lane = jax.lax.broadcasted_iota(jnp.int32, (1, HWp), 1)
    h_idx = lane // W
    w_idx = lane % W
    taps = []
    for ky in range(KS):
        dy = ky - PAD
        row_ok = jnp.logical_and(h_idx + dy >= 0, h_idx + dy < H)
        for kx in range(KS):
            dx = kx - PAD
            if dy == 0 and dx == 0:
                continue
            col_ok = jnp.logical_and(w_idx + dx >= 0, w_idx + dx < W)
            d = dy * W + dx
            taps.append((ky * KS + kx, (-d) % HWp,
                         jnp.logical_and(row_ok, col_ok)))

    t_center = (KS // 2) * KS + KS // 2
    Cin = x_ref.shape[1]
    for c in range(n_chunks):
        r0 = c * CR
        b, c0 = r0 // Cin, r0 % Cin
        xb = x_ref[b, pl.ds(c0, CR), :, :].astype(bf16).reshape(CR, H * W)
        r16 = jnp.maximum(xb, 0)
        packed = pltpu.bitcast(r16, jnp.int32)
        if HWp % 128 == 0:
            nrep = HWp // 128
            wide = lambda t: pltpu.repeat(
                dww_ref[t, pl.ds(r0, CR), :], nrep, axis=1)
        else:
            wide = lambda t: dww_ref[t, pl.ds(r0, CR), 0:1]
        acc_a = r16 * wide(t_center)
        acc_b = None
        for i, (t, shift, valid) in enumerate(taps):
            rolled = pltpu.roll(packed, shift, axis=1)
            masked = jnp.where(valid, rolled, 0)
            mb = pltpu.bitcast(masked, bf16)
            term = mb * wide(t)
            if i % 2 == 0:
                acc_a = acc_a + term
            else:
                acc_b = term if acc_b is None else acc_b + term
        full = (xb + dwb_ref[pl.ds(r0, CR), :]) + (acc_a + acc_b)
        acc_ref[pl.ds(r0, CR), :] = full

    res = (jnp.dot(wblk_ref[...], acc_ref[...], preferred_element_type=f32)
           + pwb_ref[...]).astype(out_ref.dtype)
    out_ref[...] = res.reshape(out_ref.shape)


@jax.jit
def _resblock2d_fast(x_nchw, dw_w, dw_b, pw_w, pw_b):
    N, Cin, H, W = x_nchw.shape
    Cout = pw_w.shape[1]
    HW = H * W
    HWp = HW
    R = 2 * Cin
    Ro = 2 * Cout
    n_chunks = 4 if (R % 4 == 0 and (R // 4) % 2 == 0) else 1

    f32 = jnp.float32
    bf16 = jnp.bfloat16


    dww2 = jnp.broadcast_to(
        jnp.concatenate([dw_w, dw_w], axis=1).astype(bf16)[:, :, None],
        (KS * KS, R, 128))
    dwb2 = jnp.concatenate([dw_b, dw_b]).astype(bf16)[:, None]
    wblk = jnp.kron(jnp.eye(2, dtype=f32), pw_w.T).astype(bf16)
    pwb2 = jnp.concatenate([pw_b, pw_b]).astype(f32)[:, None]

    body = functools.partial(_pair_kernel, H=H, W=W, HWp=HWp, R=R,
                             n_chunks=n_chunks)

    out4 = pl.pallas_call(
        body,
        out_shape=jax.ShapeDtypeStruct((N, Cout, H, W), x_nchw.dtype),
        grid=(N // 2,),
        in_specs=[
            pl.BlockSpec((2, Cin, H, W), lambda i: (i, 0, 0, 0)),
            pl.BlockSpec((KS * KS, R, 128), lambda i: (0, 0, 0)),
            pl.BlockSpec((R, 1), lambda i: (0, 0)),
            pl.BlockSpec((Ro, R), lambda i: (0, 0)),
            pl.BlockSpec((Ro, 1), lambda i: (0, 0)),
        ],
        out_specs=pl.BlockSpec((2, Cout, H, W), lambda i: (i, 0, 0, 0)),
        scratch_shapes=[pltpu.VMEM((R, HWp), bf16)],
        compiler_params=pltpu.CompilerParams(
            dimension_semantics=("parallel",),
            vmem_limit_bytes=48 * 1024 * 1024,
        ),
    )(x_nchw, dww2, dwb2, wblk, pwb2)

    return out4


def kernel(x, dw_w, dw_b, pw_w, pw_b):
    return _resblock2d_fast(x, dw_w, dw_b, pw_w, pw_b)


# speedup vs baseline: 1.2315x; 1.2315x over previous
import functools

import jax
import jax.numpy as jnp
from jax.experimental import pallas as pl
from jax.experimental.pallas import tpu as pltpu

KS = 5
PAD = KS // 2


def _pair_kernel(x_ref, dww_ref, dwb_ref, wblk_ref, pwb_ref, out_ref, acc_ref,
                 *, H, W, HWp, R, n_chunks):
    f32 = jnp.float32
    bf16 = jnp.bfloat16
    CR = R // n_chunks
    Ro = out_ref.shape[0]

    lane = jax.lax.broadcasted_iota(jnp.int32, (1, HWp), 1)
    h_idx = lane // W
    w_idx = lane % W
    taps = []
    for ky in range(KS):
        dy = ky - PAD
        row_ok = jnp.logical_and(h_idx + dy >= 0, h_idx + dy < H)
        for kx in range(KS):
            dx = kx - PAD
            if dy == 0 and dx == 0:
                continue
            col_ok = jnp.logical_and(w_idx + dx >= 0, w_idx + dx < W)
            d = dy * W + dx
            taps.append((ky * KS + kx, (-d) % HWp,
                         jnp.logical_and(row_ok, col_ok)))

    t_center = (KS // 2) * KS + KS // 2
    for c in range(n_chunks):
        r0 = c * CR
        xb = x_ref[pl.ds(r0, CR), :, :].astype(bf16).reshape(CR, H * W)
        r16 = jnp.maximum(xb, 0)
        packed = pltpu.bitcast(r16, jnp.int32)
        if HWp % 128 == 0:
            nrep = HWp // 128
            wide = lambda t: pltpu.repeat(
                dww_ref[t, pl.ds(r0, CR), :], nrep, axis=1)
        else:
            wide = lambda t: dww_ref[t, pl.ds(r0, CR), 0:1]
        acc_a = r16 * wide(t_center)
        acc_b = None
        for i, (t, shift, valid) in enumerate(taps):
            rolled = pltpu.roll(packed, shift, axis=1)
            masked = jnp.where(valid, rolled, 0)
            mb = pltpu.bitcast(masked, bf16)
            term = mb * wide(t)
            if i % 2 == 0:
                acc_a = acc_a + term
            else:
                acc_b = term if acc_b is None else acc_b + term
        full = (xb + dwb_ref[pl.ds(r0, CR), :]) + (acc_a + acc_b)
        acc_ref[pl.ds(r0, CR), :] = full

    res = (jnp.dot(wblk_ref[...], acc_ref[...], preferred_element_type=f32)
           + pwb_ref[...]).astype(out_ref.dtype)
    out_ref[...] = res


@jax.jit
def _resblock2d_fast(x_nchw, dw_w, dw_b, pw_w, pw_b):
    N, Cin, H, W = x_nchw.shape
    Cout = pw_w.shape[1]
    HW = H * W
    HWp = HW
    R = 2 * Cin
    Ro = 2 * Cout
    n_chunks = 4 if (R % 4 == 0 and (R // 4) % 2 == 0) else 1

    f32 = jnp.float32
    bf16 = jnp.bfloat16

    x3 = x_nchw.reshape(N * Cin, H, W)

    dww2 = jnp.broadcast_to(
        jnp.concatenate([dw_w, dw_w], axis=1).astype(bf16)[:, :, None],
        (KS * KS, R, 128))
    dwb2 = jnp.concatenate([dw_b, dw_b]).astype(bf16)[:, None]
    wblk = jnp.kron(jnp.eye(2, dtype=f32), pw_w.T).astype(bf16)
    pwb2 = jnp.concatenate([pw_b, pw_b]).astype(f32)[:, None]

    body = functools.partial(_pair_kernel, H=H, W=W, HWp=HWp, R=R,
                             n_chunks=n_chunks)

    out3 = pl.pallas_call(
        body,
        out_shape=jax.ShapeDtypeStruct((N * Cout, HWp), x_nchw.dtype),
        grid=(N // 2,),
        in_specs=[
            pl.BlockSpec((R, H, W), lambda i: (i, 0, 0)),
            pl.BlockSpec((KS * KS, R, 128), lambda i: (0, 0, 0)),
            pl.BlockSpec((R, 1), lambda i: (0, 0)),
            pl.BlockSpec((Ro, R), lambda i: (0, 0)),
            pl.BlockSpec((Ro, 1), lambda i: (0, 0)),
        ],
        out_specs=pl.BlockSpec((Ro, HWp), lambda i: (i, 0)),
        scratch_shapes=[pltpu.VMEM((R, HWp), bf16)],
        compiler_params=pltpu.CompilerParams(
            dimension_semantics=("parallel",),
            vmem_limit_bytes=48 * 1024 * 1024,
        ),
    )(x3, dww2, dwb2, wblk, pwb2)

    return out3.reshape(N, Cout, H, W)


def kernel(x, dw_w, dw_b, pw_w, pw_b):
    return _resblock2d_fast(x, dw_w, dw_b, pw_w, pw_b)


# speedup vs baseline: 1.3580x; 1.1028x over previous
import functools

import jax
import jax.numpy as jnp
from jax.experimental import pallas as pl
from jax.experimental.pallas import tpu as pltpu

KS = 5
PAD = KS // 2


def _pair_kernel(x_ref, dww_ref, dwb_ref, wblk_ref, pwb_ref, out_ref, acc_ref,
                 *, H, W, HWp, R, n_chunks):
    f32 = jnp.float32
    bf16 = jnp.bfloat16
    CR = R // n_chunks
    Ro = out_ref.shape[0]

    lane = jax.lax.broadcasted_iota(jnp.int32, (1, HWp), 1)
    h_idx = lane // W
    w_idx = lane % W
    taps = []
    for ky in range(KS):
        dy = ky - PAD
        row_ok = jnp.logical_and(h_idx + dy >= 0, h_idx + dy < H)
        for kx in range(KS):
            dx = kx - PAD
            if dy == 0 and dx == 0:
                continue
            col_ok = jnp.logical_and(w_idx + dx >= 0, w_idx + dx < W)
            d = dy * W + dx
            taps.append((ky * KS + kx, (-d) % HWp,
                         jnp.logical_and(row_ok, col_ok)))

    t_center = (KS // 2) * KS + KS // 2
    for c in range(n_chunks):
        r0 = c * CR
        xb = x_ref[pl.ds(r0, CR), :, :].astype(bf16).reshape(CR, H * W)
        r16 = jnp.maximum(xb, 0)
        packed = pltpu.bitcast(r16, jnp.int32)
        if HWp % 128 == 0:
            nrep = HWp // 128
            wide = lambda t: pltpu.repeat(
                dww_ref[t, pl.ds(r0, CR), :], nrep, axis=1)
        else:
            wide = lambda t: dww_ref[t, pl.ds(r0, CR), 0:1]
        acc_a = r16 * wide(t_center)
        acc_b = None
        for i, (t, shift, valid) in enumerate(taps):
            rolled = pltpu.roll(packed, shift, axis=1)
            masked = jnp.where(valid, rolled, 0)
            mb = pltpu.bitcast(masked, bf16)
            term = mb * wide(t)
            if i % 2 == 0:
                acc_a = acc_a + term
            else:
                acc_b = term if acc_b is None else acc_b + term
        full = (xb + dwb_ref[pl.ds(r0, CR), :]) + (acc_a + acc_b)
        acc_ref[pl.ds(r0, CR), :] = full

    res = (jnp.dot(wblk_ref[...], acc_ref[...], preferred_element_type=f32)
           + pwb_ref[...]).astype(out_ref.dtype)
    out_ref[...] = res.reshape(out_ref.shape)


@jax.jit
def _resblock2d_fast(x_nchw, dw_w, dw_b, pw_w, pw_b):
    N, Cin, H, W = x_nchw.shape
    Cout = pw_w.shape[1]
    HW = H * W
    HWp = HW
    NPAIR = 2 if N % 4 == 0 else 1
    R = NPAIR * 2 * Cin
    Ro = NPAIR * 2 * Cout
    n_chunks = R // 32 if (R % 32 == 0 and Cin % 32 == 0) else (
        4 if (R % 4 == 0 and (R // 4) % 2 == 0) else 1)

    f32 = jnp.float32
    bf16 = jnp.bfloat16

    x3 = x_nchw.reshape(N * Cin, H, W)

    nb = 2 * NPAIR
    dww2 = jnp.broadcast_to(
        jnp.concatenate([dw_w] * nb, axis=1).astype(bf16)[:, :, None],
        (KS * KS, R, 128))
    dwb2 = jnp.concatenate([dw_b] * nb).astype(bf16)[:, None]
    wblk = jnp.kron(jnp.eye(nb, dtype=f32), pw_w.T).astype(bf16)
    pwb2 = jnp.concatenate([pw_b] * nb).astype(f32)[:, None]

    body = functools.partial(_pair_kernel, H=H, W=W, HWp=HWp, R=R,
                             n_chunks=n_chunks)

    out3 = pl.pallas_call(
        body,
        out_shape=jax.ShapeDtypeStruct((N * Cout, H, W), x_nchw.dtype),
        grid=(N // (2 * NPAIR),),
        in_specs=[
            pl.BlockSpec((R, H, W), lambda i: (i, 0, 0)),
            pl.BlockSpec((KS * KS, R, 128), lambda i: (0, 0, 0)),
            pl.BlockSpec((R, 1), lambda i: (0, 0)),
            pl.BlockSpec((Ro, R), lambda i: (0, 0)),
            pl.BlockSpec((Ro, 1), lambda i: (0, 0)),
        ],
        out_specs=pl.BlockSpec((Ro, H, W), lambda i: (i, 0, 0)),
        scratch_shapes=[pltpu.VMEM((R, HWp), bf16)],
        compiler_params=pltpu.CompilerParams(
            dimension_semantics=("parallel",),
            vmem_limit_bytes=48 * 1024 * 1024,
        ),
    )(x3, dww2, dwb2, wblk, pwb2)

    return out3.reshape(N, Cout, H, W)


def kernel(x, dw_w, dw_b, pw_w, pw_b):
    return _resblock2d_fast(x, dw_w, dw_b, pw_w, pw_b)
